# Initial kernel scaffold; baseline (speedup 1.0000x reference)
import jax
import jax.numpy as jnp
from jax.experimental import pallas as pl


def kernel(x, edge_index, edge_attr, batch, W_emb, b_emb, W_msg, b_msg, W_self, b_conv, W1, b1, W2, b2):
    raise NotImplementedError("write your pallas kernel here")


# speedup vs baseline: 1.4347x; 1.4347x over previous
import functools

import jax
import jax.numpy as jnp
from jax import lax
from jax.experimental import pallas as pl
from jax.experimental.pallas import tpu as pltpu
from jax.experimental.pallas import tpu_sc as plsc

_NC, _NS = 2, 16
_CH = 128
_EB = 40
_G = 64


def _stage_a(x, W_emb, b_emb, W_msg_h, W_self):
    N, F = x.shape
    H = W_emb.shape[1]
    BLK = 1000

    def body(x_r, we_r, be_r, wmh_r, ws_r, hw0, hw1, hw2, hw3, hs_r):
        h = jnp.maximum(
            jnp.dot(x_r[...], we_r[...], preferred_element_type=jnp.float32)
            + be_r[...], 0.0)
        outs = (hw0, hw1, hw2, hw3)
        for c in range(4):
            outs[c][...] = jnp.dot(h, wmh_r[:, c * _CH:(c + 1) * _CH],
                                   preferred_element_type=jnp.float32)
        hs_r[...] = jnp.dot(h, ws_r[...], preferred_element_type=jnp.float32)

    grid = (N // BLK,)
    hw_shape = jax.ShapeDtypeStruct((N, _CH), jnp.float32)
    return pl.pallas_call(
        body,
        grid=grid,
        in_specs=[
            pl.BlockSpec((BLK, F), lambda i: (i, 0)),
            pl.BlockSpec((F, H), lambda i: (0, 0)),
            pl.BlockSpec((1, H), lambda i: (0, 0)),
            pl.BlockSpec((H, H), lambda i: (0, 0)),
            pl.BlockSpec((H, H), lambda i: (0, 0)),
        ],
        out_specs=[pl.BlockSpec((BLK, _CH), lambda i: (i, 0))] * 4
        + [pl.BlockSpec((BLK, H), lambda i: (i, 0))],
        out_shape=[hw_shape] * 4 + [jax.ShapeDtypeStruct((N, H), jnp.float32)],
    )(x, W_emb, b_emb, W_msg_h, W_self)


def _stage_a2(edge_attr, W_msg_e, b_msg):
    E, D = edge_attr.shape
    H = W_msg_e.shape[1]
    BLK = 2000

    def body(ea_r, wme_r, bm_r, o0, o1, o2, o3):
        ea = ea_r[...]
        outs = (o0, o1, o2, o3)
        for c in range(4):
            outs[c][...] = (
                jnp.dot(ea, wme_r[:, c * _CH:(c + 1) * _CH],
                        preferred_element_type=jnp.float32)
                + bm_r[:, c * _CH:(c + 1) * _CH])

    return pl.pallas_call(
        body,
        grid=(E // BLK,),
        in_specs=[
            pl.BlockSpec((BLK, D), lambda i: (i, 0)),
            pl.BlockSpec((D, H), lambda i: (0, 0)),
            pl.BlockSpec((1, H), lambda i: (0, 0)),
        ],
        out_specs=[pl.BlockSpec((BLK, _CH), lambda i: (i, 0))] * 4,
        out_shape=[jax.ShapeDtypeStruct((E, _CH), jnp.float32)] * 4,
    )(edge_attr, W_msg_e, b_msg)


def _edge_agg_sc(hw_list, eaw_list, src, dst):
    N = hw_list[0].shape[0]
    E = src.shape[0]
    e_per_tile = E // _NS
    nb = e_per_tile // _EB
    S0 = (N // _NS) // 8 * 8
    TAIL = N - S0 * _NS
    NZ = S0 // _EB
    ZREM = S0 - NZ * _EB
    mesh = plsc.VectorSubcoreMesh(core_axis_name="c", subcore_axis_name="s")

    @functools.partial(
        pl.kernel,
        out_type=[jax.ShapeDtypeStruct((N, _CH), jnp.float32)] * 4
        + [jax.ShapeDtypeStruct((N, 16), jnp.float32)],
        mesh=mesh,
        scratch_types=[
            pltpu.VMEM_SHARED((N, _CH), jnp.float32),
            pltpu.VMEM_SHARED((N, 16), jnp.float32),
            pltpu.VMEM((_EB, 16), jnp.float32),
            pltpu.VMEM((_EB, 16), jnp.float32),
            pltpu.VMEM((_EB,), jnp.int32),
            pltpu.VMEM((_EB,), jnp.int32),
            pltpu.VMEM((_EB, _CH), jnp.float32),
            pltpu.VMEM((_EB, _CH), jnp.float32),
            pltpu.SemaphoreType.DMA,
        ],
        compiler_params=pltpu.CompilerParams(use_tc_tiling_on_sc=False),
    )
    def k(hw0, hw1, hw2, hw3, ea0, ea1, ea2, ea3, src_h, dst_h,
          agg0, agg1, agg2, agg3, deg_h,
          agg_sh, deg_sh, zb16, ones_v, srcv, dstv, rows, eav, gsem):
        cid = lax.axis_index("c")
        sid = lax.axis_index("s")
        hw_h = (hw0, hw1, hw2, hw3)
        ea_h = (ea0, ea1, ea2, ea3)
        agg_h = (agg0, agg1, agg2, agg3)

        def fz16(i, c):
            zb16[i] = jnp.zeros((16,), jnp.float32)
            ones_v[i] = jnp.ones((16,), jnp.float32)
            return c
        lax.fori_loop(0, _EB, fz16, 0)

        for chunk in range(4):
            @pl.when(cid == chunk // 2)
            def _(chunk=chunk):
                def fz(i, c):
                    for q in range(_CH // 16):
                        rows[i, pl.ds(q * 16, 16)] = jnp.zeros((16,),
                                                               jnp.float32)
                    return c
                lax.fori_loop(0, _EB, fz, 0)

                for q in range(NZ):
                    pltpu.sync_copy(
                        rows, agg_sh.at[pl.ds(sid * S0 + q * _EB, _EB)])
                pltpu.sync_copy(rows.at[pl.ds(0, ZREM)],
                                agg_sh.at[pl.ds(sid * S0 + NZ * _EB, ZREM)])
                if chunk == 0:
                    for q in range(NZ):
                        pltpu.sync_copy(
                            zb16, deg_sh.at[pl.ds(sid * S0 + q * _EB, _EB)])
                    pltpu.sync_copy(
                        zb16.at[pl.ds(0, ZREM)],
                        deg_sh.at[pl.ds(sid * S0 + NZ * _EB, ZREM)])

                @pl.when(sid == _NS - 1)
                def _():
                    pltpu.sync_copy(rows.at[pl.ds(0, TAIL)],
                                    agg_sh.at[pl.ds(_NS * S0, TAIL)])
                    if chunk == 0:
                        pltpu.sync_copy(zb16.at[pl.ds(0, TAIL)],
                                        deg_sh.at[pl.ds(_NS * S0, TAIL)])
                plsc.subcore_barrier()

                base = sid * e_per_tile

                def batch(j, c):
                    e0 = base + j * _EB
                    pltpu.sync_copy(src_h.at[pl.ds(e0, _EB)], srcv)
                    pltpu.sync_copy(dst_h.at[pl.ds(e0, _EB)], dstv)
                    pltpu.async_copy(hw_h[chunk].at[srcv], rows, gsem).wait()
                    pltpu.sync_copy(ea_h[chunk].at[pl.ds(e0, _EB)], eav)

                    def comp(i, cc):
                        for q in range(_CH // 16):
                            s = pl.ds(q * 16, 16)
                            rows[i, s] = jnp.maximum(rows[i, s] + eav[i, s],
                                                     0.0)
                        return cc
                    lax.fori_loop(0, _EB, comp, 0)
                    pltpu.sync_copy(rows, agg_sh.at[dstv], add=True)
                    if chunk == 0:
                        pltpu.sync_copy(ones_v, deg_sh.at[dstv], add=True)
                    return c
                lax.fori_loop(0, nb, batch, 0)
                plsc.subcore_barrier()

                r0 = sid * S0
                pltpu.sync_copy(agg_sh.at[pl.ds(r0, S0)],
                                agg_h[chunk].at[pl.ds(r0, S0)])
                if chunk == 0:
                    pltpu.sync_copy(deg_sh.at[pl.ds(r0, S0)],
                                    deg_h.at[pl.ds(r0, S0)])

                @pl.when(sid == _NS - 1)
                def _():
                    t0 = _NS * S0
                    pltpu.sync_copy(agg_sh.at[pl.ds(t0, TAIL)],
                                    agg_h[chunk].at[pl.ds(t0, TAIL)])
                    if chunk == 0:
                        pltpu.sync_copy(deg_sh.at[pl.ds(t0, TAIL)],
                                        deg_h.at[pl.ds(t0, TAIL)])

    return k(*hw_list, *eaw_list, src, dst)


def _stage_b(agg_list, deg, hs, b_conv, batch_col):
    N, H = hs.shape
    BLK = 400
    nblk = N // BLK

    def body(a0, a1, a2, a3, deg_r, hs_r, bc_r, bt_r, sums_r, maxr, cnt_r):
        pid = pl.program_id(0)

        @pl.when(pid == 0)
        def _():
            sums_r[...] = jnp.zeros_like(sums_r)
            maxr[...] = jnp.full_like(maxr, -jnp.inf)
            cnt_r[...] = jnp.zeros_like(cnt_r)

        bt = bt_r[...]
        invdeg = 1.0 / jnp.maximum(deg_r[:, 0:1], 1.0)
        iota = lax.broadcasted_iota(jnp.int32, (BLK, _G), 1)
        mask = (bt == iota).astype(jnp.float32)
        cnt_r[...] += jnp.sum(mask, axis=0).reshape(_G, 1)
        g0 = jnp.min(bt)
        g1 = jnp.max(bt)

        aggs = (a0, a1, a2, a3)
        for c in range(4):
            part = jnp.maximum(
                aggs[c][...] * invdeg + hs_r[:, c * _CH:(c + 1) * _CH]
                + bc_r[:, c * _CH:(c + 1) * _CH], 0.0)
            sums_r[:, c * _CH:(c + 1) * _CH] += lax.dot_general(
                mask, part, (((0,), (0,)), ((), ())),
                preferred_element_type=jnp.float32)

            def mx(g, c2):
                sel = jnp.where(bt == g, part, -jnp.inf)
                m = jnp.max(sel, axis=0).reshape(1, _CH)
                g8 = pl.multiple_of((g // 8) * 8, 8)
                cur = maxr[pl.ds(g8, 8), c * _CH:(c + 1) * _CH]
                rio = lax.broadcasted_iota(jnp.int32, (8, 1), 0)
                upd = jnp.where(rio == g - g8, jnp.maximum(cur, m), cur)
                maxr[pl.ds(g8, 8), c * _CH:(c + 1) * _CH] = upd
                return c2
            lax.fori_loop(g0, g1 + 1, mx, 0)

    return pl.pallas_call(
        body,
        grid=(nblk,),
        in_specs=[pl.BlockSpec((BLK, _CH), lambda i: (i, 0))] * 4
        + [
            pl.BlockSpec((BLK, 16), lambda i: (i, 0)),
            pl.BlockSpec((BLK, H), lambda i: (i, 0)),
            pl.BlockSpec((1, H), lambda i: (0, 0)),
            pl.BlockSpec((BLK, 1), lambda i: (i, 0)),
        ],
        out_specs=[
            pl.BlockSpec((_G, H), lambda i: (0, 0)),
            pl.BlockSpec((_G, H), lambda i: (0, 0)),
            pl.BlockSpec((_G, 1), lambda i: (0, 0)),
        ],
        out_shape=[
            jax.ShapeDtypeStruct((_G, H), jnp.float32),
            jax.ShapeDtypeStruct((_G, H), jnp.float32),
            jax.ShapeDtypeStruct((_G, 1), jnp.float32),
        ],
    )(*agg_list, deg, hs, b_conv, batch_col)


def _stage_c(sums, maxacc, counts, W1, b1, W2, b2):
    H = sums.shape[1]

    def body(s_r, m_r, c_r, w1_r, b1_r, w2_r, b2_r, o_r):
        cnt = c_r[...]
        mean = s_r[...] / jnp.maximum(cnt, 1.0)
        mx = jnp.where(cnt > 0.0, m_r[...], 0.0)
        r = (jnp.dot(mean, w1_r[0:H, :], preferred_element_type=jnp.float32)
             + jnp.dot(mx, w1_r[H:2 * H, :],
                       preferred_element_type=jnp.float32)
             + b1_r[...])
        r = jnp.maximum(r, 0.0)
        o_r[...] = (jnp.dot(r, w2_r[...], preferred_element_type=jnp.float32)
                    + b2_r[...])

    return pl.pallas_call(
        body,
        out_shape=jax.ShapeDtypeStruct((_G, 1), jnp.float32),
    )(sums, maxacc, counts, W1, b1, W2, b2)


def kernel(x, edge_index, edge_attr, batch, W_emb, b_emb, W_msg, b_msg,
           W_self, b_conv, W1, b1, W2, b2):
    N, F = x.shape
    H = W_emb.shape[1]
    src = edge_index[0]
    dst = edge_index[1]

    W_msg_h = W_msg[:H]
    W_msg_e = W_msg[H:]

    *hw_list, hs = _stage_a(x, W_emb, b_emb.reshape(1, H), W_msg_h, W_self)
    eaw_list = _stage_a2(edge_attr, W_msg_e, b_msg.reshape(1, H))
    *agg_list, deg = _edge_agg_sc(hw_list, eaw_list, src, dst)
    sums, maxacc, counts = _stage_b(agg_list, deg, hs,
                                    b_conv.reshape(1, H),
                                    batch.reshape(N, 1))
    out = _stage_c(sums, maxacc, counts, W1, b1.reshape(1, H), W2,
                   b2.reshape(1, 1))
    return out.reshape(_G)


# speedup vs baseline: 3.2238x; 2.2471x over previous
import functools

import jax
import jax.numpy as jnp
from jax import lax
from jax.experimental import pallas as pl
from jax.experimental.pallas import tpu as pltpu
from jax.experimental.pallas import tpu_sc as plsc

_NC, _NS = 2, 16
_CH = 128
_EB = 40
_WB = 50
_DH = 80
_G = 64


def _stage_a(x, W_emb, b_emb, W_msg_h, W_self):
    N, F = x.shape
    H = W_emb.shape[1]
    BLK = 1000

    def body(x_r, we_r, be_r, wmh_r, ws_r, hw0, hw1, hw2, hw3, hs_r):
        h = jnp.maximum(
            jnp.dot(x_r[...], we_r[...], preferred_element_type=jnp.float32)
            + be_r[...], 0.0)
        outs = (hw0, hw1, hw2, hw3)
        for c in range(4):
            outs[c][...] = jnp.dot(h, wmh_r[:, c * _CH:(c + 1) * _CH],
                                   preferred_element_type=jnp.float32)
        hs_r[...] = jnp.dot(h, ws_r[...], preferred_element_type=jnp.float32)

    grid = (N // BLK,)
    hw_shape = jax.ShapeDtypeStruct((N, _CH), jnp.float32)
    return pl.pallas_call(
        body,
        grid=grid,
        in_specs=[
            pl.BlockSpec((BLK, F), lambda i: (i, 0)),
            pl.BlockSpec((F, H), lambda i: (0, 0)),
            pl.BlockSpec((1, H), lambda i: (0, 0)),
            pl.BlockSpec((H, H), lambda i: (0, 0)),
            pl.BlockSpec((H, H), lambda i: (0, 0)),
        ],
        out_specs=[pl.BlockSpec((BLK, _CH), lambda i: (i, 0))] * 4
        + [pl.BlockSpec((BLK, H), lambda i: (i, 0))],
        out_shape=[hw_shape] * 4 + [jax.ShapeDtypeStruct((N, H), jnp.float32)],
    )(x, W_emb, b_emb, W_msg_h, W_self)


def _stage_a2(edge_attr, W_msg_e, b_msg):
    E, D = edge_attr.shape
    H = W_msg_e.shape[1]
    BLK = 2000

    def body(ea_r, wme_r, bm_r, o0, o1, o2, o3):
        ea = ea_r[...]
        outs = (o0, o1, o2, o3)
        for c in range(4):
            outs[c][...] = (
                jnp.dot(ea, wme_r[:, c * _CH:(c + 1) * _CH],
                        preferred_element_type=jnp.float32)
                + bm_r[:, c * _CH:(c + 1) * _CH])

    return pl.pallas_call(
        body,
        grid=(E // BLK,),
        in_specs=[
            pl.BlockSpec((BLK, D), lambda i: (i, 0)),
            pl.BlockSpec((D, H), lambda i: (0, 0)),
            pl.BlockSpec((1, H), lambda i: (0, 0)),
        ],
        out_specs=[pl.BlockSpec((BLK, _CH), lambda i: (i, 0))] * 4,
        out_shape=[jax.ShapeDtypeStruct((E, _CH), jnp.float32)] * 4,
    )(edge_attr, W_msg_e, b_msg)


def _edge_agg_sc(hw_list, eaw_list, src2d, dst2d):
    N = hw_list[0].shape[0]
    NBR = src2d.shape[0]
    rows_pt = NBR // _NS
    NW = rows_pt // _WB
    S0 = (N // _NS) // 8 * 8
    TAIL = N - S0 * _NS
    NZ = S0 // _EB
    ZREM = S0 - NZ * _EB
    DR = _DH // _NS
    mesh = plsc.VectorSubcoreMesh(core_axis_name="c", subcore_axis_name="s")

    @functools.partial(
        pl.kernel,
        out_type=[jax.ShapeDtypeStruct((N, _CH), jnp.float32)] * 4
        + [jax.ShapeDtypeStruct((_DH, 128), jnp.float32)],
        mesh=mesh,
        scratch_types=[
            pltpu.VMEM_SHARED((N, _CH), jnp.float32),
            pltpu.VMEM((_EB, _CH), jnp.float32),
            pltpu.VMEM((_EB, _CH), jnp.float32),
            pltpu.VMEM((_EB, _CH), jnp.float32),
            pltpu.VMEM((_EB, _CH), jnp.float32),
            pltpu.VMEM((_WB, _EB), jnp.int32),
            pltpu.VMEM((_WB, _EB), jnp.int32),
            pltpu.VMEM((_DH, 128), jnp.float32),
            pltpu.SemaphoreType.DMA,
            pltpu.SemaphoreType.DMA,
            pltpu.SemaphoreType.DMA,
            pltpu.SemaphoreType.DMA,
            pltpu.SemaphoreType.DMA,
            pltpu.SemaphoreType.DMA,
        ],
        compiler_params=pltpu.CompilerParams(use_tc_tiling_on_sc=False,
                                             needs_layout_passes=False),
    )
    def k(hw0, hw1, hw2, hw3, ea0, ea1, ea2, ea3, src_h, dst_h,
          agg0, agg1, agg2, agg3, deg_h,
          agg_sh, rows0, rows1, eav0, eav1, srcb, dstb, ldeg,
          gs0, gs1, es0, es1, ss0, ss1):
        cid = lax.axis_index("c")
        sid = lax.axis_index("s")
        hw_h = (hw0, hw1, hw2, hw3)
        ea_h = (ea0, ea1, ea2, ea3)
        agg_h = (agg0, agg1, agg2, agg3)
        rows = (rows0, rows1)
        eav = (eav0, eav1)
        gsem = (gs0, gs1)
        esem = (es0, es1)
        ssem = (ss0, ss1)

        def zld(i, c):
            for q in range(8):
                ldeg[i, pl.ds(q * 16, 16)] = jnp.zeros((16,), jnp.float32)
            return c
        lax.fori_loop(0, _DH, zld, 0)

        iota16 = lax.broadcasted_iota(jnp.int32, (16,), 0)
        m_hi8 = iota16 >= 8
        ones16 = jnp.ones((16,), jnp.float32)

        for chunk in range(4):
            @pl.when(cid == chunk // 2)
            def _(chunk=chunk):
                hw = hw_h[chunk]
                eaw = ea_h[chunk]
                agg_o = agg_h[chunk]

                def fz(i, c):
                    for q in range(_CH // 16):
                        rows[0][i, pl.ds(q * 16, 16)] = jnp.zeros(
                            (16,), jnp.float32)
                    return c
                lax.fori_loop(0, _EB, fz, 0)

                for q in range(NZ):
                    pltpu.sync_copy(
                        rows[0], agg_sh.at[pl.ds(sid * S0 + q * _EB, _EB)])
                pltpu.sync_copy(rows[0].at[pl.ds(0, ZREM)],
                                agg_sh.at[pl.ds(sid * S0 + NZ * _EB, ZREM)])

                @pl.when(sid == _NS - 1)
                def _():
                    pltpu.sync_copy(rows[0].at[pl.ds(0, TAIL)],
                                    agg_sh.at[pl.ds(_NS * S0, TAIL)])
                plsc.subcore_barrier()

                base_row = sid * rows_pt

                def relu_add(u):
                    def comp(i2, cc):
                        for rr in range(4):
                            i = i2 * 4 + rr
                            for q in range(_CH // 16):
                                s = pl.ds(q * 16, 16)
                                rows[u][i, s] = jnp.maximum(
                                    rows[u][i, s] + eav[u][i, s], 0.0)
                        return cc
                    lax.fori_loop(0, _EB // 4, comp, 0)

                def window(w, c):
                    w0 = base_row + w * _WB
                    pltpu.sync_copy(src_h.at[pl.ds(w0, _WB)], srcb)
                    pltpu.sync_copy(dst_h.at[pl.ds(w0, _WB)], dstb)
                    if chunk == 0:
                        def hist(i, cc):
                            for off, msk in ((0, None), (16, None),
                                             (24, m_hi8)):
                                idxv = dstb[i, pl.ds(off, 16)]
                                hr = lax.shift_right_logical(idxv, 7)
                                hc = lax.bitwise_and(idxv, 127)
                                if msk is None:
                                    plsc.addupdate_scatter(
                                        ldeg, [hr, hc], ones16)
                                else:
                                    plsc.addupdate_scatter(
                                        ldeg, [hr, hc], ones16, mask=msk)
                            return cc
                        lax.fori_loop(0, _WB, hist, 0)

                    def issue(b, u):
                        pltpu.async_copy(hw.at[srcb.at[b]], rows[u], gsem[u])
                        e0 = (w0 + b) * _EB
                        pltpu.async_copy(eaw.at[pl.ds(e0, _EB)], eav[u],
                                         esem[u])

                    def drain(sem, dst):
                        pltpu.make_async_copy(eaw.at[pl.ds(0, _EB)], dst,
                                              sem).wait()

                    issue(0, 0)
                    issue(1, 1)

                    def steady(kk, cc):
                        for u in (0, 1):
                            b = 2 * kk + u
                            drain(gsem[u], rows[u])
                            drain(esem[u], eav[u])
                            relu_add(u)
                            pltpu.async_copy(rows[u], agg_sh.at[dstb.at[b]],
                                             ssem[u], add=True)
                        for u in (0, 1):
                            b = 2 * kk + u
                            drain(ssem[u], rows[u])
                            issue(b + 2, u)
                        return cc
                    lax.fori_loop(0, _WB // 2 - 1, steady, 0)

                    for u in (0, 1):
                        drain(gsem[u], rows[u])
                        drain(esem[u], eav[u])
                        relu_add(u)
                        pltpu.async_copy(rows[u],
                                         agg_sh.at[dstb.at[_WB - 2 + u]],
                                         ssem[u], add=True)
                    for u in (0, 1):
                        drain(ssem[u], rows[u])
                    return c
                lax.fori_loop(0, NW, window, 0)
                plsc.subcore_barrier()

                r0 = sid * S0
                pltpu.sync_copy(agg_sh.at[pl.ds(r0, S0)],
                                agg_o.at[pl.ds(r0, S0)])

                @pl.when(sid == _NS - 1)
                def _():
                    t0 = _NS * S0
                    pltpu.sync_copy(agg_sh.at[pl.ds(t0, TAIL)],
                                    agg_o.at[pl.ds(t0, TAIL)])

                if chunk == 0:
                    plsc.subcore_barrier()
                    pltpu.sync_copy(ldeg, agg_sh.at[pl.ds(_DH * sid, _DH)])
                    plsc.subcore_barrier()
                    r5 = DR * sid
                    pltpu.sync_copy(agg_sh.at[pl.ds(r5, DR)],
                                    rows[0].at[pl.ds(0, DR)])
                    for t in range(1, _NS):
                        pltpu.sync_copy(agg_sh.at[pl.ds(_DH * t + r5, DR)],
                                        eav[0].at[pl.ds(0, DR)])
                        for i in range(DR):
                            for q in range(8):
                                s = pl.ds(q * 16, 16)
                                rows[0][i, s] = rows[0][i, s] + eav[0][i, s]
                    pltpu.sync_copy(rows[0].at[pl.ds(0, DR)],
                                    deg_h.at[pl.ds(r5, DR)])
                    plsc.subcore_barrier()

    return k(*hw_list, *eaw_list, src2d, dst2d)


def _stage_b(agg_list, deg, hs, b_conv, batch_col):
    N, H = hs.shape
    BLK = 400
    nblk = N // BLK

    def body(a0, a1, a2, a3, deg_r, hs_r, bc_r, bt_r, sums_r, maxr, cnt_r):
        pid = pl.program_id(0)

        @pl.when(pid == 0)
        def _():
            sums_r[...] = jnp.zeros_like(sums_r)
            maxr[...] = jnp.full_like(maxr, -jnp.inf)
            cnt_r[...] = jnp.zeros_like(cnt_r)

        bt = bt_r[...]
        invdeg = 1.0 / jnp.maximum(deg_r[...], 1.0)
        iota = lax.broadcasted_iota(jnp.int32, (BLK, _G), 1)
        mask = (bt == iota).astype(jnp.float32)
        cnt_r[...] += jnp.sum(mask, axis=0).reshape(_G, 1)
        g0 = jnp.min(bt)
        g1 = jnp.max(bt)

        aggs = (a0, a1, a2, a3)
        for c in range(4):
            part = jnp.maximum(
                aggs[c][...] * invdeg + hs_r[:, c * _CH:(c + 1) * _CH]
                + bc_r[:, c * _CH:(c + 1) * _CH], 0.0)
            sums_r[:, c * _CH:(c + 1) * _CH] += lax.dot_general(
                mask, part, (((0,), (0,)), ((), ())),
                preferred_element_type=jnp.float32)

            def mx(g, c2):
                sel = jnp.where(bt == g, part, -jnp.inf)
                m = jnp.max(sel, axis=0).reshape(1, _CH)
                g8 = pl.multiple_of((g // 8) * 8, 8)
                cur = maxr[pl.ds(g8, 8), c * _CH:(c + 1) * _CH]
                rio = lax.broadcasted_iota(jnp.int32, (8, 1), 0)
                upd = jnp.where(rio == g - g8, jnp.maximum(cur, m), cur)
                maxr[pl.ds(g8, 8), c * _CH:(c + 1) * _CH] = upd
                return c2
            lax.fori_loop(g0, g1 + 1, mx, 0)

    return pl.pallas_call(
        body,
        grid=(nblk,),
        in_specs=[pl.BlockSpec((BLK, _CH), lambda i: (i, 0))] * 4
        + [
            pl.BlockSpec((BLK, 1), lambda i: (i, 0)),
            pl.BlockSpec((BLK, H), lambda i: (i, 0)),
            pl.BlockSpec((1, H), lambda i: (0, 0)),
            pl.BlockSpec((BLK, 1), lambda i: (i, 0)),
        ],
        out_specs=[
            pl.BlockSpec((_G, H), lambda i: (0, 0)),
            pl.BlockSpec((_G, H), lambda i: (0, 0)),
            pl.BlockSpec((_G, 1), lambda i: (0, 0)),
        ],
        out_shape=[
            jax.ShapeDtypeStruct((_G, H), jnp.float32),
            jax.ShapeDtypeStruct((_G, H), jnp.float32),
            jax.ShapeDtypeStruct((_G, 1), jnp.float32),
        ],
    )(*agg_list, deg, hs, b_conv, batch_col)


def _stage_c(sums, maxacc, counts, W1, b1, W2, b2):
    H = sums.shape[1]

    def body(s_r, m_r, c_r, w1_r, b1_r, w2_r, b2_r, o_r):
        cnt = c_r[...]
        mean = s_r[...] / jnp.maximum(cnt, 1.0)
        mx = jnp.where(cnt > 0.0, m_r[...], 0.0)
        r = (jnp.dot(mean, w1_r[0:H, :], preferred_element_type=jnp.float32)
             + jnp.dot(mx, w1_r[H:2 * H, :],
                       preferred_element_type=jnp.float32)
             + b1_r[...])
        r = jnp.maximum(r, 0.0)
        o_r[...] = (jnp.dot(r, w2_r[...], preferred_element_type=jnp.float32)
                    + b2_r[...])

    return pl.pallas_call(
        body,
        out_shape=jax.ShapeDtypeStruct((_G, 1), jnp.float32),
    )(sums, maxacc, counts, W1, b1, W2, b2)


def kernel(x, edge_index, edge_attr, batch, W_emb, b_emb, W_msg, b_msg,
           W_self, b_conv, W1, b1, W2, b2):
    N, F = x.shape
    H = W_emb.shape[1]
    src = edge_index[0]
    dst = edge_index[1]

    W_msg_h = W_msg[:H]
    W_msg_e = W_msg[H:]

    E = src.shape[0]
    *hw_list, hs = _stage_a(x, W_emb, b_emb.reshape(1, H), W_msg_h, W_self)
    eaw_list = _stage_a2(edge_attr, W_msg_e, b_msg.reshape(1, H))
    *agg_list, deg_hist = _edge_agg_sc(hw_list, eaw_list,
                                       src.reshape(E // _EB, _EB),
                                       dst.reshape(E // _EB, _EB))
    deg = deg_hist.reshape(-1)[:N].reshape(N, 1)
    sums, maxacc, counts = _stage_b(agg_list, deg, hs,
                                    b_conv.reshape(1, H),
                                    batch.reshape(N, 1))
    out = _stage_c(sums, maxacc, counts, W1, b1.reshape(1, H), W2,
                   b2.reshape(1, 1))
    return out.reshape(_G)
